# baseline (speedup 1.0000x reference)
import functools
import math

import numpy as np
import jax
import jax.numpy as jnp
from jax import lax
from jax.experimental import pallas as pl
from jax.experimental.pallas import tpu as pltpu
from jax.experimental.pallas import tpu_sc as plsc

_VOCAB = 100000
_DIM = 768
_B, _L = 4, 4096
_SCALE = math.sqrt(_DIM)
_LANES = 16
_DV = _DIM // _LANES

_NC, _NS = 2, 16
_NW = _NC * _NS
_N = _B * _L
_PPW = _L // _NW
_NIDX = _B * _PPW
_CP = 4
_CR = _B * _CP
_T = _PPW // _CP
_R = 8
_D = 6
_S = _T // _R


def _pe_table() -> np.ndarray:
    position = np.arange(_L, dtype=np.float32)[:, None]
    div_term = np.exp(
        np.arange(0, _DIM, 2, dtype=np.float32) * (-math.log(10000.0) / _DIM)
    )
    pe = np.zeros((_L, _DIM), dtype=np.float32)
    pe[:, 0::2] = np.sin(position * div_term)
    pe[:, 1::2] = np.cos(position * div_term)
    return pe


_PE = _pe_table()


@functools.partial(
    pl.kernel,
    mesh=plsc.VectorSubcoreMesh(core_axis_name="c", subcore_axis_name="s"),
    out_type=jax.ShapeDtypeStruct((_N, _DIM), jnp.float32),
    scratch_types=(
        [pltpu.VMEM((_NIDX,), jnp.int32)]
        + [pltpu.VMEM((_CR, _DIM), jnp.float32) for _ in range(_R)]
        + [pltpu.VMEM((_CP, _DIM), jnp.float32) for _ in range(_R)]
        + [pltpu.SemaphoreType.DMA for _ in range(3 * _R)]
    ),
)
def _embed(table_hbm, idx_hbm, pe_hbm, out_hbm, idx_v, *bufs):
    rows = bufs[0:_R]
    pe_v = bufs[_R:2 * _R]
    gsem = bufs[2 * _R:3 * _R]
    psem = bufs[3 * _R:4 * _R]
    osem = bufs[4 * _R:5 * _R]

    cid = lax.axis_index("c")
    sid = lax.axis_index("s")
    wid = cid * _NS + sid
    pbase = wid * _PPW

    pltpu.sync_copy(idx_hbm.at[pl.ds(wid * _NIDX, _NIDX)], idx_v)

    def issue(t_, q_):
        pltpu.async_copy(
            table_hbm.at[idx_v.at[pl.ds(t_ * _CR, _CR)]], rows[q_], gsem[q_]
        )
        pltpu.async_copy(
            pe_hbm.at[pl.ds(pbase + t_ * _CP, _CP)], pe_v[q_], psem[q_]
        )

    def wait_in(q_):
        pltpu.make_async_copy(table_hbm.at[pl.ds(0, _CR)], rows[q_], gsem[q_]).wait()
        pltpu.make_async_copy(pe_hbm.at[pl.ds(0, _CP)], pe_v[q_], psem[q_]).wait()

    def wait_out(q_):
        pltpu.make_async_copy(rows[q_], out_hbm.at[pl.ds(0, _CR)], osem[q_]).wait()

    def compute(q_):
        def j_body(j, acc):
            for cv in range(_DV):
                sl = pl.ds(cv * _LANES, _LANES)
                pv = pe_v[q_][j, sl]
                for b in range(_B):
                    r = b * _CP + j
                    rows[q_][r, sl] = rows[q_][r, sl] * _SCALE + pv
            return acc
        lax.fori_loop(0, _CP, j_body, 0)

    def issue_out(t_, q_):
        for b in range(_B):
            pltpu.async_copy(
                rows[q_].at[pl.ds(b * _CP, _CP)],
                out_hbm.at[pl.ds(b * _L + pbase + t_ * _CP, _CP)],
                osem[q_],
            )

    for i in range(_D):
        issue(i, i)

    def s_body(s, acc):
        for k in range(_R):
            t = s * _R + k
            q = k
            wait_in(q)
            compute(q)
            issue_out(t, q)
            qn = (k + _D) % _R
            if k < _R - _D:
                @pl.when(s > 0)
                def _w(qn_=qn):
                    wait_out(qn_)
                issue(t + _D, qn)
            else:
                wait_out(qn)
                @pl.when(s < _S - 1)
                def _i(t_=t + _D, qn_=qn):
                    issue(t_, qn_)
        return acc

    lax.fori_loop(0, _S, s_body, 0)
    for i in range(_T - 2, _T):
        wait_out(i % _R)


def kernel(x, table):
    xf = (x.astype(jnp.int32)
          .reshape(_B, _NW, _T, _CP)
          .transpose(1, 2, 0, 3)
          .reshape(-1))
    pe = jnp.asarray(_PE)
    out = _embed(table, xf, pe)
    return out.reshape(_B, _L, _DIM)


# speedup vs baseline: 1.1256x; 1.1256x over previous
import functools
import math

import numpy as np
import jax
import jax.numpy as jnp
from jax import lax
from jax.experimental import pallas as pl
from jax.experimental.pallas import tpu as pltpu
from jax.experimental.pallas import tpu_sc as plsc

_VOCAB = 100000
_DIM = 768
_B, _L = 4, 4096
_SCALE = math.sqrt(_DIM)
_LANES = 16
_DV = _DIM // _LANES

_NC, _NS = 2, 16
_NW = _NC * _NS
_N = _B * _L
_PPW = _L // _NW
_NIDX = _B * _PPW
_CP = 8
_CR = _B * _CP
_T = _PPW // _CP
_R = 4
_D = 2
_S = _T // _R


def _pe_table() -> np.ndarray:
    position = np.arange(_L, dtype=np.float32)[:, None]
    div_term = np.exp(
        np.arange(0, _DIM, 2, dtype=np.float32) * (-math.log(10000.0) / _DIM)
    )
    pe = np.zeros((_L, _DIM), dtype=np.float32)
    pe[:, 0::2] = np.sin(position * div_term)
    pe[:, 1::2] = np.cos(position * div_term)
    return pe


_PE = _pe_table()


@functools.partial(
    pl.kernel,
    mesh=plsc.VectorSubcoreMesh(core_axis_name="c", subcore_axis_name="s"),
    out_type=jax.ShapeDtypeStruct((_N, _DIM), jnp.float32),
    scratch_types=(
        [pltpu.VMEM((_NIDX,), jnp.int32)]
        + [pltpu.VMEM((_CR, _DIM), jnp.float32) for _ in range(_R)]
        + [pltpu.VMEM((_CP, _DIM), jnp.float32) for _ in range(_R)]
        + [pltpu.SemaphoreType.DMA for _ in range(3 * _R)]
    ),
)
def _embed(table_hbm, idx_hbm, pe_hbm, out_hbm, idx_v, *bufs):
    rows = bufs[0:_R]
    pe_v = bufs[_R:2 * _R]
    gsem = bufs[2 * _R:3 * _R]
    psem = bufs[3 * _R:4 * _R]
    osem = bufs[4 * _R:5 * _R]

    cid = lax.axis_index("c")
    sid = lax.axis_index("s")
    wid = cid * _NS + sid
    pbase = wid * _PPW

    pltpu.sync_copy(idx_hbm.at[pl.ds(wid * _NIDX, _NIDX)], idx_v)

    def issue(t_, q_):
        pltpu.async_copy(
            table_hbm.at[idx_v.at[pl.ds(t_ * _CR, _CR)]], rows[q_], gsem[q_]
        )
        pltpu.async_copy(
            pe_hbm.at[pl.ds(pbase + t_ * _CP, _CP)], pe_v[q_], psem[q_]
        )

    def wait_in(q_):
        pltpu.make_async_copy(table_hbm.at[pl.ds(0, _CR)], rows[q_], gsem[q_]).wait()
        pltpu.make_async_copy(pe_hbm.at[pl.ds(0, _CP)], pe_v[q_], psem[q_]).wait()

    def wait_out(q_):
        pltpu.make_async_copy(rows[q_], out_hbm.at[pl.ds(0, _CR)], osem[q_]).wait()

    def compute(q_):
        def j_body(j, acc):
            for cv in range(_DV):
                sl = pl.ds(cv * _LANES, _LANES)
                pv = pe_v[q_][j, sl]
                for b in range(_B):
                    r = b * _CP + j
                    rows[q_][r, sl] = rows[q_][r, sl] * _SCALE + pv
            return acc
        lax.fori_loop(0, _CP, j_body, 0)

    def issue_out(t_, q_):
        for b in range(_B):
            pltpu.async_copy(
                rows[q_].at[pl.ds(b * _CP, _CP)],
                out_hbm.at[pl.ds(b * _L + pbase + t_ * _CP, _CP)],
                osem[q_],
            )

    for i in range(_D):
        issue(i, i)

    def s_body(s, acc):
        for k in range(_R):
            t = s * _R + k
            q = k
            wait_in(q)
            compute(q)
            issue_out(t, q)
            qn = (k + _D) % _R
            if k < _R - _D:
                @pl.when(s > 0)
                def _w(qn_=qn):
                    wait_out(qn_)
                issue(t + _D, qn)
            else:
                wait_out(qn)
                @pl.when(s < _S - 1)
                def _i(t_=t + _D, qn_=qn):
                    issue(t_, qn_)
        return acc

    lax.fori_loop(0, _S, s_body, 0)
    for i in range(_T - 2, _T):
        wait_out(i % _R)


def kernel(x, table):
    xf = (x.astype(jnp.int32)
          .reshape(_B, _NW, _T, _CP)
          .transpose(1, 2, 0, 3)
          .reshape(-1))
    pe = jnp.asarray(_PE)
    out = _embed(table, xf, pe)
    return out.reshape(_B, _L, _DIM)


# speedup vs baseline: 1.2258x; 1.0890x over previous
import functools
import math

import numpy as np
import jax
import jax.numpy as jnp
from jax import lax
from jax.experimental import pallas as pl
from jax.experimental.pallas import tpu as pltpu
from jax.experimental.pallas import tpu_sc as plsc

_VOCAB = 100000
_DIM = 768
_B, _L = 4, 4096
_SCALE = math.sqrt(_DIM)
_LANES = 16
_DG = _DIM // (2 * _LANES)

_NC, _NS = 2, 16
_NW = _NC * _NS
_N = _B * _L
_PPW = _L // _NW
_NIDX = _B * _PPW
_CP = 8
_CR = _B * _CP
_T = _PPW // _CP
_R = 4
_D = 2
_S = _T // _R


def _pe_table_bf16() -> np.ndarray:
    position = np.arange(_L, dtype=np.float32)[:, None]
    div_term = np.exp(
        np.arange(0, _DIM, 2, dtype=np.float32) * (-math.log(10000.0) / _DIM)
    )
    pe = np.zeros((_L, _DIM), dtype=np.float32)
    pe[:, 0::2] = np.sin(position * div_term)
    pe[:, 1::2] = np.cos(position * div_term)
    pe = pe.reshape(_L, _DG, 2, _LANES).transpose(0, 1, 3, 2).reshape(_L, _DIM)
    return np.asarray(pe.astype(jnp.bfloat16)).view(np.int32).reshape(-1)


_PE = _pe_table_bf16()


@functools.partial(
    pl.kernel,
    mesh=plsc.VectorSubcoreMesh(core_axis_name="c", subcore_axis_name="s"),
    out_type=jax.ShapeDtypeStruct((_N, _DIM), jnp.float32),
    scratch_types=(
        [pltpu.VMEM((_NIDX,), jnp.int32)]
        + [pltpu.VMEM((_CR, _DIM), jnp.float32) for _ in range(_R)]
        + [pltpu.VMEM((_CP * _DIM // 2,), jnp.int32) for _ in range(_R)]
        + [pltpu.SemaphoreType.DMA for _ in range(3 * _R)]
    ),
)
def _embed(table_hbm, idx_hbm, pe_hbm, out_hbm, idx_v, *bufs):
    rows = bufs[0:_R]
    pe_v = bufs[_R:2 * _R]
    gsem = bufs[2 * _R:3 * _R]
    psem = bufs[3 * _R:4 * _R]
    osem = bufs[4 * _R:5 * _R]

    cid = lax.axis_index("c")
    sid = lax.axis_index("s")
    wid = cid * _NS + sid
    pbase = wid * _PPW

    pltpu.sync_copy(idx_hbm.at[pl.ds(wid * _NIDX, _NIDX)], idx_v)

    def issue(t_, q_):
        pltpu.async_copy(
            table_hbm.at[idx_v.at[pl.ds(t_ * _CR, _CR)]], rows[q_], gsem[q_]
        )
        pltpu.async_copy(
            pe_hbm.at[pl.ds((pbase + t_ * _CP) * (_DIM // 2), _CP * (_DIM // 2))],
            pe_v[q_], psem[q_],
        )

    def wait_in(q_):
        pltpu.make_async_copy(table_hbm.at[pl.ds(0, _CR)], rows[q_], gsem[q_]).wait()
        pltpu.make_async_copy(pe_hbm.at[pl.ds(0, _CP * _DIM // 2)], pe_v[q_], psem[q_]).wait()

    def wait_out(q_):
        pltpu.make_async_copy(rows[q_], out_hbm.at[pl.ds(0, _CR)], osem[q_]).wait()

    def compute(q_):
        def j_body(j, acc):
            for g in range(_DG):
                w = pe_v[q_][pl.ds(j * (_DIM // 2) + g * _LANES, _LANES)]
                pa = lax.bitcast_convert_type(w << 16, jnp.float32)
                pb = lax.bitcast_convert_type(w & jnp.int32(-65536), jnp.float32)
                sla = pl.ds(g * 2 * _LANES, _LANES)
                slb = pl.ds(g * 2 * _LANES + _LANES, _LANES)
                for b in range(_B):
                    r = b * _CP + j
                    rows[q_][r, sla] = rows[q_][r, sla] * _SCALE + pa
                    rows[q_][r, slb] = rows[q_][r, slb] * _SCALE + pb
            return acc
        lax.fori_loop(0, _CP, j_body, 0)

    def issue_out(t_, q_):
        for b in range(_B):
            pltpu.async_copy(
                rows[q_].at[pl.ds(b * _CP, _CP)],
                out_hbm.at[pl.ds(b * _L + pbase + t_ * _CP, _CP)],
                osem[q_],
            )

    for i in range(_D):
        issue(i, i)

    def s_body(s, acc):
        for k in range(_R):
            t = s * _R + k
            q = k
            wait_in(q)
            compute(q)
            issue_out(t, q)
            qn = (k + _D) % _R
            if k < _R - _D:
                @pl.when(s > 0)
                def _w(qn_=qn):
                    wait_out(qn_)
                issue(t + _D, qn)
            else:
                wait_out(qn)
                @pl.when(s < _S - 1)
                def _i(t_=t + _D, qn_=qn):
                    issue(t_, qn_)
        return acc

    lax.fori_loop(0, _S, s_body, 0)
    for i in range(_T - 2, _T):
        wait_out(i % _R)


def kernel(x, table):
    xf = (x.astype(jnp.int32)
          .reshape(_B, _NW, _T, _CP)
          .transpose(1, 2, 0, 3)
          .reshape(-1))
    pe = jnp.asarray(_PE)
    out = _embed(table, xf, pe)
    return out.reshape(_B, _L, _DIM)
